# Initial kernel scaffold; baseline (speedup 1.0000x reference)
import jax
import jax.numpy as jnp
from jax.experimental import pallas as pl


def kernel(x, edge_index, c1W1, c1b1, c1W2, c1b2, c2W1, c2b1, c2W2, c2b2, fcW, fcb):
    raise NotImplementedError("write your pallas kernel here")


# speedup vs baseline: 8.1278x; 8.1278x over previous
import functools

import jax
import jax.numpy as jnp
from jax import lax
from jax.experimental import pallas as pl
from jax.experimental.pallas import tpu as pltpu
from jax.experimental.pallas import tpu_sc as plsc

N_NODES = 10000
IN_CH = 128
HID = 16
NUM_CLASSES = 40

NC = 2
NS = 16
NW = NC * NS
CHUNK = 128
NCHUNK = 80
E_PAD = NW * NCHUNK * CHUNK
ACC_ROWS = 10240
RPT = ACC_ROWS // NS
JUNK_ROW = N_NODES

BR = 1000
GRID = N_NODES // BR


def _scatter_add_call(feat, src3, dst3, zeros, F):
    mesh = plsc.VectorSubcoreMesh(core_axis_name="c", subcore_axis_name="s")

    @functools.partial(
        pl.kernel,
        out_type=(
            jax.ShapeDtypeStruct((ACC_ROWS, F), jnp.float32),
            jax.ShapeDtypeStruct((ACC_ROWS, F), jnp.float32),
        ),
        mesh=mesh,
        scratch_types=[
            pltpu.VMEM((NCHUNK, CHUNK), jnp.int32),
            pltpu.VMEM((NCHUNK, CHUNK), jnp.int32),
            pltpu.VMEM((CHUNK, F), jnp.float32),
            pltpu.VMEM_SHARED((ACC_ROWS, F), jnp.float32),
            pltpu.SemaphoreType.DMA,
        ],
        compiler_params=pltpu.CompilerParams(use_tc_tiling_on_sc=False),
    )
    def k(feat_hbm, src_hbm, dst_hbm, zero_hbm, out0, out1,
          srcv, dstv, rows, acc, sem):
        cid = lax.axis_index("c")
        sid = lax.axis_index("s")
        wid = cid * NS + sid
        pltpu.sync_copy(zero_hbm, acc.at[pl.ds(sid * RPT, RPT)])
        pltpu.sync_copy(src_hbm.at[wid], srcv)
        pltpu.sync_copy(dst_hbm.at[wid], dstv)
        plsc.subcore_barrier()

        def body(j, carry):
            pltpu.async_copy(feat_hbm.at[srcv.at[j]], rows, sem).wait()
            pltpu.sync_copy(rows, acc.at[dstv.at[j]], add=True)
            return carry

        lax.fori_loop(0, NCHUNK, body, 0)
        plsc.subcore_barrier()

        @pl.when(cid == 0)
        def _():
            pltpu.sync_copy(acc.at[pl.ds(sid * RPT, RPT)],
                            out0.at[pl.ds(sid * RPT, RPT)])

        @pl.when(cid == 1)
        def _():
            pltpu.sync_copy(acc.at[pl.ds(sid * RPT, RPT)],
                            out1.at[pl.ds(sid * RPT, RPT)])

    return k(feat, src3, dst3, zeros)


def _proj_call(x, w):
    def body(x_ref, w_ref, o_ref):
        o_ref[...] = jnp.dot(x_ref[...], w_ref[...],
                             preferred_element_type=jnp.float32)

    return pl.pallas_call(
        body,
        grid=(GRID,),
        in_specs=[
            pl.BlockSpec((BR, IN_CH), lambda i: (i, 0)),
            pl.BlockSpec((IN_CH, 2 * HID), lambda i: (0, 0)),
        ],
        out_specs=pl.BlockSpec((BR, 2 * HID), lambda i: (i, 0)),
        out_shape=jax.ShapeDtypeStruct((N_NODES, 2 * HID), jnp.float32),
    )(x, w)


def _mlp1_call(p, a0, a1, b1, W2, b2):
    def body(p_ref, a0_ref, a1_ref, b1_ref, W2_ref, b2_ref, o_ref):
        t = p_ref[...] + a0_ref[...] + a1_ref[...] + b1_ref[...]
        t = jnp.maximum(t, 0.0)
        h = jnp.dot(t, W2_ref[...], preferred_element_type=jnp.float32)
        o_ref[...] = jnp.maximum(h + b2_ref[...], 0.0)

    return pl.pallas_call(
        body,
        grid=(GRID,),
        in_specs=[
            pl.BlockSpec((BR, 2 * HID), lambda i: (i, 0)),
            pl.BlockSpec((BR, 2 * HID), lambda i: (i, 0)),
            pl.BlockSpec((BR, 2 * HID), lambda i: (i, 0)),
            pl.BlockSpec((1, 2 * HID), lambda i: (0, 0)),
            pl.BlockSpec((2 * HID, HID), lambda i: (0, 0)),
            pl.BlockSpec((1, HID), lambda i: (0, 0)),
        ],
        out_specs=pl.BlockSpec((BR, HID), lambda i: (i, 0)),
        out_shape=jax.ShapeDtypeStruct((N_NODES, HID), jnp.float32),
    )(p, a0, a1, b1, W2, b2)


def _mlp2_call(h1, a0, a1, W1, b1, W2, b2, fcW, fcb):
    def body(h_ref, a0_ref, a1_ref, W1_ref, b1_ref, W2_ref, b2_ref,
             fcW_ref, fcb_ref, o_ref):
        t = h_ref[...] + a0_ref[...] + a1_ref[...]
        u = jnp.dot(t, W1_ref[...], preferred_element_type=jnp.float32)
        u = jnp.maximum(u + b1_ref[...], 0.0)
        v = jnp.dot(u, W2_ref[...], preferred_element_type=jnp.float32)
        v = jnp.maximum(v + b2_ref[...], 0.0)
        o_ref[...] = jnp.dot(v, fcW_ref[...],
                             preferred_element_type=jnp.float32) + fcb_ref[...]

    return pl.pallas_call(
        body,
        grid=(GRID,),
        in_specs=[
            pl.BlockSpec((BR, HID), lambda i: (i, 0)),
            pl.BlockSpec((BR, HID), lambda i: (i, 0)),
            pl.BlockSpec((BR, HID), lambda i: (i, 0)),
            pl.BlockSpec((HID, 2 * HID), lambda i: (0, 0)),
            pl.BlockSpec((1, 2 * HID), lambda i: (0, 0)),
            pl.BlockSpec((2 * HID, HID), lambda i: (0, 0)),
            pl.BlockSpec((1, HID), lambda i: (0, 0)),
            pl.BlockSpec((HID, NUM_CLASSES), lambda i: (0, 0)),
            pl.BlockSpec((1, NUM_CLASSES), lambda i: (0, 0)),
        ],
        out_specs=pl.BlockSpec((BR, NUM_CLASSES), lambda i: (i, 0)),
        out_shape=jax.ShapeDtypeStruct((N_NODES, NUM_CLASSES), jnp.float32),
    )(h1, a0, a1, W1, b1, W2, b2, fcW, fcb)


def kernel(x, edge_index, c1W1, c1b1, c1W2, c1b2,
           c2W1, c2b1, c2W2, c2b2, fcW, fcb):
    E = edge_index.shape[1]
    ei = edge_index.astype(jnp.int32)
    pad = E_PAD - E
    src = jnp.concatenate([ei[0], jnp.zeros((pad,), jnp.int32)])
    dst = jnp.concatenate([ei[1], jnp.full((pad,), JUNK_ROW, jnp.int32)])
    src3 = src.reshape(NW, NCHUNK, CHUNK)
    dst3 = dst.reshape(NW, NCHUNK, CHUNK)
    zeros32 = jnp.zeros((RPT, 2 * HID), jnp.float32)
    zeros16 = jnp.zeros((RPT, HID), jnp.float32)

    p = _proj_call(x, c1W1)
    a0, a1 = _scatter_add_call(p, src3, dst3, zeros32, 2 * HID)
    h1 = _mlp1_call(p, a0, a1, c1b1.reshape(1, -1), c1W2,
                    c1b2.reshape(1, -1))
    g0, g1 = _scatter_add_call(h1, src3, dst3, zeros16, HID)
    logits = _mlp2_call(h1, g0, g1, c2W1, c2b1.reshape(1, -1), c2W2,
                        c2b2.reshape(1, -1), fcW, fcb.reshape(1, -1))
    return logits


# speedup vs baseline: 8.7218x; 1.0731x over previous
import functools

import jax
import jax.numpy as jnp
from jax import lax
from jax.experimental import pallas as pl
from jax.experimental.pallas import tpu as pltpu
from jax.experimental.pallas import tpu_sc as plsc

N_NODES = 10000
IN_CH = 128
HID = 16
NUM_CLASSES = 40

NC = 2
NS = 16
NW = NC * NS
CHUNK = 128
NCHUNK = 80
E_PAD = NW * NCHUNK * CHUNK
ACC_ROWS = 10240
RPT = ACC_ROWS // NS
JUNK_ROW = N_NODES

BR = 1000
GRID = N_NODES // BR


def _scatter_add_call(feat, src3, dst3, zeros, F):
    mesh = plsc.VectorSubcoreMesh(core_axis_name="c", subcore_axis_name="s")

    @functools.partial(
        pl.kernel,
        out_type=(
            jax.ShapeDtypeStruct((ACC_ROWS, F), jnp.float32),
            jax.ShapeDtypeStruct((ACC_ROWS, F), jnp.float32),
        ),
        mesh=mesh,
        scratch_types=[
            pltpu.VMEM((NCHUNK, CHUNK), jnp.int32),
            pltpu.VMEM((NCHUNK, CHUNK), jnp.int32),
            pltpu.VMEM((2, CHUNK, F), jnp.float32),
            pltpu.VMEM_SHARED((ACC_ROWS, F), jnp.float32),
            pltpu.SemaphoreType.DMA,
        ],
        compiler_params=pltpu.CompilerParams(use_tc_tiling_on_sc=False),
    )
    def k(feat_hbm, src_hbm, dst_hbm, zero_hbm, out0, out1,
          srcv, dstv, rows, acc, sem):
        cid = lax.axis_index("c")
        sid = lax.axis_index("s")
        wid = cid * NS + sid
        pltpu.sync_copy(zero_hbm, acc.at[pl.ds(sid * RPT, RPT)])
        pltpu.sync_copy(src_hbm.at[wid], srcv)
        pltpu.sync_copy(dst_hbm.at[wid], dstv)
        plsc.subcore_barrier()

        def body(j, carry):
            cur = lax.rem(j, 2)
            prv = lax.rem(j + 1, 2)
            cp = pltpu.async_copy(feat_hbm.at[srcv.at[j]], rows.at[cur], sem)

            @pl.when(j > 0)
            def _():
                pltpu.sync_copy(rows.at[prv], acc.at[dstv.at[j - 1]],
                                add=True)

            cp.wait()
            return carry

        lax.fori_loop(0, NCHUNK, body, 0)
        pltpu.sync_copy(rows.at[(NCHUNK - 1) % 2],
                        acc.at[dstv.at[NCHUNK - 1]], add=True)
        plsc.subcore_barrier()

        @pl.when(cid == 0)
        def _():
            pltpu.sync_copy(acc.at[pl.ds(sid * RPT, RPT)],
                            out0.at[pl.ds(sid * RPT, RPT)])

        @pl.when(cid == 1)
        def _():
            pltpu.sync_copy(acc.at[pl.ds(sid * RPT, RPT)],
                            out1.at[pl.ds(sid * RPT, RPT)])

    return k(feat, src3, dst3, zeros)


def _proj_call(x, w):
    def body(x_ref, w_ref, o_ref):
        o_ref[...] = jnp.dot(x_ref[...], w_ref[...],
                             preferred_element_type=jnp.float32)

    return pl.pallas_call(
        body,
        grid=(GRID,),
        in_specs=[
            pl.BlockSpec((BR, IN_CH), lambda i: (i, 0)),
            pl.BlockSpec((IN_CH, 2 * HID), lambda i: (0, 0)),
        ],
        out_specs=pl.BlockSpec((BR, 2 * HID), lambda i: (i, 0)),
        out_shape=jax.ShapeDtypeStruct((N_NODES, 2 * HID), jnp.float32),
    )(x, w)


def _mlp1_call(p, a0, a1, b1, W2, b2):
    def body(p_ref, a0_ref, a1_ref, b1_ref, W2_ref, b2_ref, o_ref):
        t = p_ref[...] + a0_ref[...] + a1_ref[...] + b1_ref[...]
        t = jnp.maximum(t, 0.0)
        h = jnp.dot(t, W2_ref[...], preferred_element_type=jnp.float32)
        o_ref[...] = jnp.maximum(h + b2_ref[...], 0.0)

    return pl.pallas_call(
        body,
        grid=(GRID,),
        in_specs=[
            pl.BlockSpec((BR, 2 * HID), lambda i: (i, 0)),
            pl.BlockSpec((BR, 2 * HID), lambda i: (i, 0)),
            pl.BlockSpec((BR, 2 * HID), lambda i: (i, 0)),
            pl.BlockSpec((1, 2 * HID), lambda i: (0, 0)),
            pl.BlockSpec((2 * HID, HID), lambda i: (0, 0)),
            pl.BlockSpec((1, HID), lambda i: (0, 0)),
        ],
        out_specs=pl.BlockSpec((BR, HID), lambda i: (i, 0)),
        out_shape=jax.ShapeDtypeStruct((N_NODES, HID), jnp.float32),
    )(p, a0, a1, b1, W2, b2)


def _mlp2_call(h1, a0, a1, W1, b1, W2, b2, fcW, fcb):
    def body(h_ref, a0_ref, a1_ref, W1_ref, b1_ref, W2_ref, b2_ref,
             fcW_ref, fcb_ref, o_ref):
        t = h_ref[...] + a0_ref[...] + a1_ref[...]
        u = jnp.dot(t, W1_ref[...], preferred_element_type=jnp.float32)
        u = jnp.maximum(u + b1_ref[...], 0.0)
        v = jnp.dot(u, W2_ref[...], preferred_element_type=jnp.float32)
        v = jnp.maximum(v + b2_ref[...], 0.0)
        o_ref[...] = jnp.dot(v, fcW_ref[...],
                             preferred_element_type=jnp.float32) + fcb_ref[...]

    return pl.pallas_call(
        body,
        grid=(GRID,),
        in_specs=[
            pl.BlockSpec((BR, HID), lambda i: (i, 0)),
            pl.BlockSpec((BR, HID), lambda i: (i, 0)),
            pl.BlockSpec((BR, HID), lambda i: (i, 0)),
            pl.BlockSpec((HID, 2 * HID), lambda i: (0, 0)),
            pl.BlockSpec((1, 2 * HID), lambda i: (0, 0)),
            pl.BlockSpec((2 * HID, HID), lambda i: (0, 0)),
            pl.BlockSpec((1, HID), lambda i: (0, 0)),
            pl.BlockSpec((HID, NUM_CLASSES), lambda i: (0, 0)),
            pl.BlockSpec((1, NUM_CLASSES), lambda i: (0, 0)),
        ],
        out_specs=pl.BlockSpec((BR, NUM_CLASSES), lambda i: (i, 0)),
        out_shape=jax.ShapeDtypeStruct((N_NODES, NUM_CLASSES), jnp.float32),
    )(h1, a0, a1, W1, b1, W2, b2, fcW, fcb)


def kernel(x, edge_index, c1W1, c1b1, c1W2, c1b2,
           c2W1, c2b1, c2W2, c2b2, fcW, fcb):
    E = edge_index.shape[1]
    ei = edge_index.astype(jnp.int32)
    pad = E_PAD - E
    src = jnp.concatenate([ei[0], jnp.zeros((pad,), jnp.int32)])
    dst = jnp.concatenate([ei[1], jnp.full((pad,), JUNK_ROW, jnp.int32)])
    src3 = src.reshape(NW, NCHUNK, CHUNK)
    dst3 = dst.reshape(NW, NCHUNK, CHUNK)
    zeros32 = jnp.zeros((RPT, 2 * HID), jnp.float32)
    zeros16 = jnp.zeros((RPT, HID), jnp.float32)

    p = _proj_call(x, c1W1)
    a0, a1 = _scatter_add_call(p, src3, dst3, zeros32, 2 * HID)
    h1 = _mlp1_call(p, a0, a1, c1b1.reshape(1, -1), c1W2,
                    c1b2.reshape(1, -1))
    g0, g1 = _scatter_add_call(h1, src3, dst3, zeros16, HID)
    logits = _mlp2_call(h1, g0, g1, c2W1, c2b1.reshape(1, -1), c2W2,
                        c2b2.reshape(1, -1), fcW, fcb.reshape(1, -1))
    return logits


# speedup vs baseline: 10.4429x; 1.1973x over previous
import functools

import jax
import jax.numpy as jnp
from jax import lax
from jax.experimental import pallas as pl
from jax.experimental.pallas import tpu as pltpu
from jax.experimental.pallas import tpu_sc as plsc

N_NODES = 10000
IN_CH = 128
HID = 16
NUM_CLASSES = 40

NC = 2
NS = 16
NW = NC * NS
CHUNK = 128
NCHUNK = 80
E_PAD = NW * NCHUNK * CHUNK
ACC_ROWS = 10240
RPT = ACC_ROWS // NS
JUNK_ROW = N_NODES

BR = 1000
GRID = N_NODES // BR


def _scatter_add_call(feat, src3, dst3, zeros, F):
    mesh = plsc.VectorSubcoreMesh(core_axis_name="c", subcore_axis_name="s")

    @functools.partial(
        pl.kernel,
        out_type=(
            jax.ShapeDtypeStruct((ACC_ROWS, F), jnp.float32),
            jax.ShapeDtypeStruct((ACC_ROWS, F), jnp.float32),
        ),
        mesh=mesh,
        scratch_types=[
            pltpu.VMEM((NCHUNK, CHUNK), jnp.int32),
            pltpu.VMEM((NCHUNK, CHUNK), jnp.int32),
            pltpu.VMEM((4, CHUNK, F), jnp.float32),
            pltpu.VMEM_SHARED((ACC_ROWS, F), jnp.float32),
            pltpu.SemaphoreType.DMA,
        ],
        compiler_params=pltpu.CompilerParams(use_tc_tiling_on_sc=False),
    )
    def k(feat_hbm, src_hbm, dst_hbm, zero_hbm, out0, out1,
          srcv, dstv, rows, acc, sem):
        cid = lax.axis_index("c")
        sid = lax.axis_index("s")
        wid = cid * NS + sid
        pltpu.sync_copy(zero_hbm, acc.at[pl.ds(sid * RPT, RPT)])
        pltpu.sync_copy(src_hbm.at[wid], srcv)
        pltpu.sync_copy(dst_hbm.at[wid], dstv)
        plsc.subcore_barrier()

        for j0 in range(3):
            pltpu.async_copy(feat_hbm.at[srcv.at[j0]], rows.at[j0], sem)

        def body(j, carry):
            @pl.when(j + 3 < NCHUNK)
            def _():
                jn = jnp.minimum(j + 3, NCHUNK - 1)
                pltpu.async_copy(feat_hbm.at[srcv.at[jn]],
                                 rows.at[lax.rem(jn, 4)], sem)

            cur = lax.rem(j, 4)
            pltpu.make_async_copy(feat_hbm.at[srcv.at[j]], rows.at[cur],
                                  sem).wait()
            pltpu.sync_copy(rows.at[cur], acc.at[dstv.at[j]], add=True)
            return carry

        lax.fori_loop(0, NCHUNK, body, 0)
        plsc.subcore_barrier()

        @pl.when(cid == 0)
        def _():
            pltpu.sync_copy(acc.at[pl.ds(sid * RPT, RPT)],
                            out0.at[pl.ds(sid * RPT, RPT)])

        @pl.when(cid == 1)
        def _():
            pltpu.sync_copy(acc.at[pl.ds(sid * RPT, RPT)],
                            out1.at[pl.ds(sid * RPT, RPT)])

    return k(feat, src3, dst3, zeros)


def _proj_call(x, w):
    def body(x_ref, w_ref, o_ref):
        o_ref[...] = jnp.dot(x_ref[...], w_ref[...],
                             preferred_element_type=jnp.float32)

    return pl.pallas_call(
        body,
        grid=(GRID,),
        in_specs=[
            pl.BlockSpec((BR, IN_CH), lambda i: (i, 0)),
            pl.BlockSpec((IN_CH, 2 * HID), lambda i: (0, 0)),
        ],
        out_specs=pl.BlockSpec((BR, 2 * HID), lambda i: (i, 0)),
        out_shape=jax.ShapeDtypeStruct((N_NODES, 2 * HID), jnp.float32),
    )(x, w)


def _mlp1_call(p, a0, a1, b1, W2, b2):
    def body(p_ref, a0_ref, a1_ref, b1_ref, W2_ref, b2_ref, o_ref):
        t = p_ref[...] + a0_ref[...] + a1_ref[...] + b1_ref[...]
        t = jnp.maximum(t, 0.0)
        h = jnp.dot(t, W2_ref[...], preferred_element_type=jnp.float32)
        o_ref[...] = jnp.maximum(h + b2_ref[...], 0.0)

    return pl.pallas_call(
        body,
        grid=(GRID,),
        in_specs=[
            pl.BlockSpec((BR, 2 * HID), lambda i: (i, 0)),
            pl.BlockSpec((BR, 2 * HID), lambda i: (i, 0)),
            pl.BlockSpec((BR, 2 * HID), lambda i: (i, 0)),
            pl.BlockSpec((1, 2 * HID), lambda i: (0, 0)),
            pl.BlockSpec((2 * HID, HID), lambda i: (0, 0)),
            pl.BlockSpec((1, HID), lambda i: (0, 0)),
        ],
        out_specs=pl.BlockSpec((BR, HID), lambda i: (i, 0)),
        out_shape=jax.ShapeDtypeStruct((N_NODES, HID), jnp.float32),
    )(p, a0, a1, b1, W2, b2)


def _mlp2_call(h1, a0, a1, W1, b1, W2, b2, fcW, fcb):
    def body(h_ref, a0_ref, a1_ref, W1_ref, b1_ref, W2_ref, b2_ref,
             fcW_ref, fcb_ref, o_ref):
        t = h_ref[...] + a0_ref[...] + a1_ref[...]
        u = jnp.dot(t, W1_ref[...], preferred_element_type=jnp.float32)
        u = jnp.maximum(u + b1_ref[...], 0.0)
        v = jnp.dot(u, W2_ref[...], preferred_element_type=jnp.float32)
        v = jnp.maximum(v + b2_ref[...], 0.0)
        o_ref[...] = jnp.dot(v, fcW_ref[...],
                             preferred_element_type=jnp.float32) + fcb_ref[...]

    return pl.pallas_call(
        body,
        grid=(GRID,),
        in_specs=[
            pl.BlockSpec((BR, HID), lambda i: (i, 0)),
            pl.BlockSpec((BR, HID), lambda i: (i, 0)),
            pl.BlockSpec((BR, HID), lambda i: (i, 0)),
            pl.BlockSpec((HID, 2 * HID), lambda i: (0, 0)),
            pl.BlockSpec((1, 2 * HID), lambda i: (0, 0)),
            pl.BlockSpec((2 * HID, HID), lambda i: (0, 0)),
            pl.BlockSpec((1, HID), lambda i: (0, 0)),
            pl.BlockSpec((HID, NUM_CLASSES), lambda i: (0, 0)),
            pl.BlockSpec((1, NUM_CLASSES), lambda i: (0, 0)),
        ],
        out_specs=pl.BlockSpec((BR, NUM_CLASSES), lambda i: (i, 0)),
        out_shape=jax.ShapeDtypeStruct((N_NODES, NUM_CLASSES), jnp.float32),
    )(h1, a0, a1, W1, b1, W2, b2, fcW, fcb)


def kernel(x, edge_index, c1W1, c1b1, c1W2, c1b2,
           c2W1, c2b1, c2W2, c2b2, fcW, fcb):
    E = edge_index.shape[1]
    ei = edge_index.astype(jnp.int32)
    pad = E_PAD - E
    src = jnp.concatenate([ei[0], jnp.zeros((pad,), jnp.int32)])
    dst = jnp.concatenate([ei[1], jnp.full((pad,), JUNK_ROW, jnp.int32)])
    src3 = src.reshape(NW, NCHUNK, CHUNK)
    dst3 = dst.reshape(NW, NCHUNK, CHUNK)
    zeros32 = jnp.zeros((RPT, 2 * HID), jnp.float32)
    zeros16 = jnp.zeros((RPT, HID), jnp.float32)

    p = _proj_call(x, c1W1)
    a0, a1 = _scatter_add_call(p, src3, dst3, zeros32, 2 * HID)
    h1 = _mlp1_call(p, a0, a1, c1b1.reshape(1, -1), c1W2,
                    c1b2.reshape(1, -1))
    g0, g1 = _scatter_add_call(h1, src3, dst3, zeros16, HID)
    logits = _mlp2_call(h1, g0, g1, c2W1, c2b1.reshape(1, -1), c2W2,
                        c2b2.reshape(1, -1), fcW, fcb.reshape(1, -1))
    return logits
